# Initial kernel scaffold; baseline (speedup 1.0000x reference)
import jax
import jax.numpy as jnp
from jax.experimental import pallas as pl


def kernel(logits, labels):
    raise NotImplementedError("write your pallas kernel here")


# speedup vs baseline: 2.4085x; 2.4085x over previous
import functools

import jax
import jax.numpy as jnp
from jax import lax
from jax.experimental import pallas as pl
from jax.experimental.pallas import tpu as pltpu

_NBINS = 15


def _ece_body(nrows, nblocks, logits_ref, labels_ref, bs_ref, bv_ref,
              sce_ref, acc_ref, g_ref, s_ref, a_ref):
    i = pl.program_id(0)
    R, C = logits_ref.shape
    x = logits_ref[...]
    m = jnp.max(x, axis=1, keepdims=True)
    e = jnp.exp(x - m)
    p = e / jnp.sum(e, axis=1, keepdims=True)

    lab = labels_ref[...]
    iota = lax.broadcasted_iota(jnp.int32, (R, C), 1)
    lab_oh = (iota == lab).astype(jnp.float32)

    pm = jnp.max(p, axis=1, keepdims=True)
    choice = jnp.min(jnp.where(p == pm, iota, C), axis=1, keepdims=True)
    eqf = (choice == lab).astype(jnp.float32)

    conf_lab = jnp.sum(p * lab_oh, axis=1, keepdims=True)
    bv = bv_ref[...]
    ecat = jnp.concatenate(
        [(conf_lab > bv).astype(jnp.float32), eqf, jnp.ones_like(eqf)],
        axis=1)
    a_blk = lax.dot_general(ecat, lab_oh, (((0,), (0,)), ((), ())),
                            preferred_element_type=jnp.float32)

    glist, slist = [], []
    for k in range(_NBINS + 1):
        t = bs_ref[0, k]
        mk = (p > t).astype(jnp.float32)
        glist.append(jnp.sum(mk, axis=0, keepdims=True))
        slist.append(jnp.sum(mk * p, axis=0, keepdims=True))
    g_blk = jnp.concatenate(glist, axis=0)
    s_blk = jnp.concatenate(slist, axis=0)

    @pl.when(i == 0)
    def _init():
        g_ref[...] = jnp.zeros_like(g_ref)
        s_ref[...] = jnp.zeros_like(s_ref)
        a_ref[...] = jnp.zeros_like(a_ref)

    g_ref[...] += g_blk
    s_ref[...] += s_blk
    a_ref[...] += a_blk

    @pl.when(i == nblocks - 1)
    def _finalize():
        gm = g_ref[...]
        sm = s_ref[...]
        am = a_ref[...]
        counts = gm[0:_NBINS, :] - gm[1:_NBINS + 1, :]
        confs = sm[0:_NBINS, :] - sm[1:_NBINS + 1, :]
        accn = am[0:_NBINS, :] - am[1:_NBINS + 1, :]
        prop = counts / float(nrows)
        safe = jnp.maximum(counts, 1.0)
        contrib = jnp.where(counts > 0.0,
                            jnp.abs(confs / safe - accn / safe) * prop, 0.0)
        sce_ref[...] = jnp.sum(contrib, axis=0, keepdims=True)
        acc_ref[...] = am[_NBINS + 1:_NBINS + 2, :] / am[_NBINS + 2:_NBINS + 3, :]


def kernel(logits, labels):
    N, C = logits.shape
    R = 2000 if N % 2000 == 0 else N
    nblocks = N // R
    bounds = jnp.linspace(0.0, 1.0, _NBINS + 1).reshape(1, _NBINS + 1)
    labels2 = labels.reshape(N, 1)

    out = pl.pallas_call(
        functools.partial(_ece_body, N, nblocks),
        grid=(nblocks,),
        in_specs=[
            pl.BlockSpec((R, C), lambda i: (i, 0)),
            pl.BlockSpec((R, 1), lambda i: (i, 0)),
            pl.BlockSpec(memory_space=pltpu.SMEM),
            pl.BlockSpec((1, _NBINS + 1), lambda i: (0, 0)),
        ],
        out_specs=[
            pl.BlockSpec((1, C), lambda i: (0, 0)),
            pl.BlockSpec((1, C), lambda i: (0, 0)),
        ],
        out_shape=[
            jax.ShapeDtypeStruct((1, C), jnp.float32),
            jax.ShapeDtypeStruct((1, C), jnp.float32),
        ],
        scratch_shapes=[
            pltpu.VMEM((_NBINS + 1, C), jnp.float32),
            pltpu.VMEM((_NBINS + 1, C), jnp.float32),
            pltpu.VMEM((_NBINS + 3, C), jnp.float32),
        ],
    )(logits, labels2, bounds, bounds)
    return (out[0].reshape(C), out[1].reshape(C))
